# baseline (speedup 1.0000x reference)
import jax
import jax.numpy as jnp
from jax.experimental import pallas as pl
from jax.experimental.pallas import tpu as pltpu

_BM = 400


def _support_kernel(features_ref, w_ref, out_ref):
    out_ref[...] = jnp.dot(
        features_ref[...], w_ref[...], preferred_element_type=jnp.float32
    )


def _spmm_kernel(support_ref, a_ref, out_ref):
    out_ref[...] = jnp.tanh(
        jnp.dot(a_ref[...], support_ref[...], preferred_element_type=jnp.float32)
    )


def kernel(features, A, W):
    n, d_in = features.shape
    d_out = W.shape[1]
    support = pl.pallas_call(
        _support_kernel,
        out_shape=jax.ShapeDtypeStruct((n, d_out), jnp.float32),
    )(features, W)
    return pl.pallas_call(
        _spmm_kernel,
        grid=(n // _BM,),
        in_specs=[
            pl.BlockSpec((n, d_out), lambda i: (0, 0)),
            pl.BlockSpec((_BM, n), lambda i: (i, 0)),
        ],
        out_specs=pl.BlockSpec((_BM, d_out), lambda i: (i, 0)),
        out_shape=jax.ShapeDtypeStruct((n, d_out), jnp.float32),
        compiler_params=pltpu.CompilerParams(dimension_semantics=("parallel",)),
    )(support, A)
